# baseline (speedup 1.0000x reference)
import functools

import jax
import jax.numpy as jnp
from jax.experimental import pallas as pl

_NUM_CORR_LAYERS = 10
_CORR_ALPHA = 0.979
_NUM_SM_LAYERS = 10
_SM_ALPHA = 0.756


def _final_add(y_ref, e_ref, o_ref):
    o_ref[...] = y_ref[...] + e_ref[...]


def _label_prop(edge_index, y, num_layers, alpha, post_step, n_nodes):
    src = edge_index[0]
    dst = edge_index[1]
    degs = jnp.clip(jnp.bincount(dst, length=n_nodes).astype(jnp.float32), 1.0, None)
    norm = (degs ** -0.5)[:, None]
    last = (1.0 - alpha) * y
    for _ in range(num_layers):
        y = norm * y
        agg = jax.ops.segment_sum(y[src], dst, num_segments=n_nodes)
        y = alpha * agg * norm
        y = post_step(last + y)
    return y


def kernel(y_soft, y_true, edge_index, train_nid, val_nid, test_nid, n_nodes):
    n = train_nid.shape[0] + val_nid.shape[0] + test_nid.shape[0]
    c = y_soft.shape[1]
    numel = train_nid.shape[0]
    y_true_oh = jax.nn.one_hot(y_true, c, dtype=y_soft.dtype)

    error = jnp.zeros((n, c), y_soft.dtype)
    error = error.at[train_nid].set(y_true_oh - y_soft[:numel])
    sm_err = _label_prop(edge_index, error, _NUM_CORR_LAYERS, _CORR_ALPHA,
                         lambda x: jnp.clip(x, -1.0, 1.0), n)
    sigma = jnp.abs(error[train_nid]).sum() / numel
    scale = sigma / jnp.abs(sm_err).sum(axis=1, keepdims=True)
    scale = jnp.where(jnp.isinf(scale) | (scale > 1000.0), 1.0, scale)
    all_nid = jnp.concatenate([train_nid, val_nid, test_nid], axis=0)
    result = y_soft + scale * sm_err[all_nid]
    result = jnp.where(jnp.isnan(result), y_soft, result)

    y_all = jnp.zeros((n, c), y_soft.dtype)
    y_all = y_all.at[all_nid].set(result)
    y_all = y_all.at[train_nid].set(y_true_oh)
    out = _label_prop(edge_index, y_all, _NUM_SM_LAYERS, _SM_ALPHA,
                      lambda x: jnp.clip(x, 0.0, 1.0), n)
    out = out[all_nid]

    zero = jnp.zeros_like(out)
    grid = 100
    blk = out.shape[0] // grid
    return pl.pallas_call(
        _final_add,
        grid=(grid,),
        in_specs=[pl.BlockSpec((blk, c), lambda i: (i, 0)),
                  pl.BlockSpec((blk, c), lambda i: (i, 0))],
        out_specs=pl.BlockSpec((blk, c), lambda i: (i, 0)),
        out_shape=jax.ShapeDtypeStruct(out.shape, out.dtype),
    )(out, zero)


# speedup vs baseline: 6.2745x; 6.2745x over previous
import jax
import jax.numpy as jnp
from jax import lax
from jax.experimental import pallas as pl
from jax.experimental.pallas import tpu as pltpu
from jax.experimental.pallas import tpu_sc as plsc

N, E, C = 100000, 1600000, 40
PL = 16
CPL = 10
NPL = 4
NP = 100096
TN = NP // 16
UC = 136
NUC = TN // UC
EB = 800
ET = E // 16
NEB = ET // EB
IB1 = 120
NTB1 = 60000 // IB1
SLOT1 = 32
IB2 = 800
NIB2 = N // IB2
NTB2 = 60000 // IB2
CORR_ALPHA, SM_ALPHA = 0.979, 0.756
NLAYERS = 10
NTRAIN = 60000


def _iota():
    return lax.iota(jnp.int32, 16)


def _rsqrt(x):
    i = plsc.bitcast(x, jnp.int32)
    y = plsc.bitcast(jnp.full((16,), 0x5F3759DF, jnp.int32) - (i >> 1),
                     jnp.float32)
    for _ in range(3):
        y = y * (1.5 - 0.5 * x * y * y)
    return y


def _floop(nvec, f):
    def body(v, carry):
        f(v)
        return carry
    lax.fori_loop(0, nvec, body, 0)


def _fill2d(ref, rows, width, val):
    v16 = jnp.full((16,), val, jnp.float32)

    def f(v):
        k = v * 16 + _iota()
        plsc.store_scatter(ref, [k // width, k % width], v16)
    _floop(rows * width // 16, f)


def _prop_layers(c, s, alpha, lo, hi, ysrcs, ydsts, lastb, final_dst,
                 srce, dste, norm, acc, ebuf, sidx, didx, nchunk,
                 achunk, cb1, cb2, zbuf):
    it = _iota()
    r0 = s * TN
    for l in range(NLAYERS):
        ysrc = ysrcs[l]
        ydst = ydsts[l]
        final = l == NLAYERS - 1
        for sp in range(2):
            p = 2 * c + sp

            def eb(b, cy):
                off = s * ET + b * EB
                pltpu.sync_copy(srce.at[pl.ds(off, EB)], sidx)
                pltpu.sync_copy(dste.at[pl.ds(off, EB)], didx)
                pltpu.sync_copy(ysrc.at[p].at[sidx], ebuf)
                pltpu.sync_copy(ebuf, acc.at[didx], add=True)
                return cy
            lax.fori_loop(0, NEB, eb, 0)
            plsc.subcore_barrier()

            def up(u, cy):
                rb = r0 + u * UC
                pltpu.sync_copy(acc.at[pl.ds(rb, UC), :], achunk)
                pltpu.sync_copy(zbuf, acc.at[pl.ds(rb, UC), :])
                pltpu.sync_copy(lastb.at[p, pl.ds(rb, UC), :], cb1)
                pltpu.sync_copy(norm.at[c, pl.ds(rb, UC)], nchunk)

                def f(v):
                    k2 = v * 16 + it
                    i = k2 // CPL
                    cc = k2 % CPL
                    a = plsc.load_gather(achunk, [i, cc])
                    lv = plsc.load_gather(cb1, [i, cc])
                    nv = plsc.load_gather(nchunk, [i])
                    y = jnp.clip(lv + alpha * nv * a, lo, hi)
                    outv = y if final else nv * y
                    plsc.store_scatter(cb2, [i, cc], outv)
                _floop(UC * CPL // 16, f)
                tgt = final_dst if final else ydst
                pltpu.sync_copy(cb2, tgt.at[p, pl.ds(rb, UC), :])
                return cy
            lax.fori_loop(0, NUC, up, 0)
            plsc.subcore_barrier()


def _corr_body(ysoft, ytrue, tnid, srce, dste,
               norm_o, sm_o, yfa, yfb, lastb,
               acc, ebuf, sidx, didx, ibuf, tyt, ysbuf, nrows, nchunk,
               achunk, cb1, cb2, zbuf):
    c = lax.axis_index("c")
    s = lax.axis_index("s")
    r0 = s * TN
    it = _iota()

    _fill2d(zbuf, UC, PL, 0.0)
    _fill2d(cb2, UC, PL, 0.0)
    _fill2d(ebuf, EB, PL, 1.0)

    def zacc(u, cy):
        pltpu.sync_copy(zbuf, acc.at[pl.ds(r0 + u * UC, UC), :])
        return cy
    lax.fori_loop(0, NUC, zacc, 0)
    plsc.subcore_barrier()

    def degb(b, cy):
        off = s * ET + b * EB
        pltpu.sync_copy(dste.at[pl.ds(off, EB)], didx)
        pltpu.sync_copy(ebuf, acc.at[didx], add=True)
        return cy
    lax.fori_loop(0, NEB, degb, 0)
    plsc.subcore_barrier()

    def nrm(u, cy):
        rb = r0 + u * UC
        pltpu.sync_copy(acc.at[pl.ds(rb, UC), :], achunk)
        pltpu.sync_copy(zbuf, acc.at[pl.ds(rb, UC), :])

        def f(v):
            i = v * 16 + it
            m = i < UC
            d = plsc.load_gather(achunk,
                                 [jnp.minimum(i, UC - 1),
                                  jnp.zeros((16,), jnp.int32)])
            nv = _rsqrt(jnp.maximum(d, 1.0))
            plsc.store_scatter(nchunk, [i], nv, mask=m)
        _floop((UC + 15) // 16, f)
        pltpu.sync_copy(nchunk, norm_o.at[c, pl.ds(rb, UC)])
        return cy
    lax.fori_loop(0, NUC, nrm, 0)

    def zyf(u, cy):
        rb = r0 + u * UC
        for sp in range(2):
            p = 2 * c + sp
            pltpu.sync_copy(zbuf, yfa.at[p, pl.ds(rb, UC), :])
            pltpu.sync_copy(zbuf, lastb.at[p, pl.ds(rb, UC), :])
        return cy
    lax.fori_loop(0, NUC, zyf, 0)
    plsc.subcore_barrier()

    r1 = ebuf.at[pl.ds(0, IB1), :]
    r2 = ebuf.at[pl.ds(IB1, IB1), :]

    def slot(k, cy):
        g = k * 16 + s

        @pl.when(g < NTB1)
        def _():
            off = g * IB1
            pltpu.sync_copy(tnid.at[pl.ds(off, IB1)], ibuf)
            pltpu.sync_copy(ytrue.at[pl.ds(off, IB1)], tyt)
            pltpu.sync_copy(ysoft.at[pl.ds(off, IB1), :], ysbuf)
            pltpu.sync_copy(norm_o.at[c].at[ibuf], nrows)
            for sp in range(2):
                p = 2 * c + sp

                def f(v):
                    k2 = v * 16 + it
                    i = k2 // PL
                    cc = k2 % PL
                    cls = jnp.minimum(cc + p * CPL, C - 1)
                    yt = plsc.load_gather(tyt, [i])
                    oh = jnp.where(yt == cls, 1.0, 0.0)
                    ys = plsc.load_gather(ysbuf, [i, cls])
                    nv = plsc.load_gather(nrows, [i])
                    err = jnp.where(cc < CPL, oh - ys, 0.0)
                    plsc.store_scatter(r1, [i, cc], nv * err)
                    plsc.store_scatter(r2, [i, cc], (1.0 - CORR_ALPHA) * err)
                _floop(IB1 * PL // 16, f)
                pltpu.sync_copy(r1, yfa.at[p].at[ibuf])
                pltpu.sync_copy(r2, lastb.at[p].at[ibuf])
        return cy
    lax.fori_loop(0, SLOT1, slot, 0)
    plsc.subcore_barrier()

    ysrcs = [yfa if l % 2 == 0 else yfb for l in range(NLAYERS)]
    ydsts = [yfb if l % 2 == 0 else yfa for l in range(NLAYERS)]
    _prop_layers(c, s, CORR_ALPHA, -1.0, 1.0, ysrcs, ydsts, lastb, sm_o,
                 srce, dste, norm_o, acc, ebuf, sidx, didx, nchunk,
                 achunk, cb1, cb2, zbuf)


def _assemble_body(ysoft, ytrue, anid, sm, norm,
                   yfs, lasts,
                   sig_sp, ysbuf, tyt, ibuf, sj0, sj1, sj2, sj3, nrows, scl,
                   r1, r2, sg16, sg1, zpad):
    c = lax.axis_index("c")
    s = lax.axis_index("s")
    it = _iota()
    sj = [sj0, sj1, sj2, sj3]

    _fill2d(zpad, NP - N, PL, 0.0)

    def sigk(k, tot):
        g = k * 16 + s
        off = jnp.minimum(g, NTB2 - 1) * IB2
        pltpu.sync_copy(ytrue.at[pl.ds(off, IB2)], tyt)
        pltpu.sync_copy(ysoft.at[pl.ds(off, IB2), :], ysbuf)

        def f(v, a):
            k2 = v * 16 + it
            i = k2 // C
            cc = k2 % C
            ys = plsc.load_gather(ysbuf, [i, cc])
            yt = plsc.load_gather(tyt, [i])
            oh = jnp.where(yt == cc, 1.0, 0.0)
            return a + jnp.abs(oh - ys)
        a16 = lax.fori_loop(0, IB2 * C // 16, f,
                            jnp.zeros((16,), jnp.float32))
        return tot + jnp.where(g < NTB2, jnp.sum(a16), 0.0)
    tot = lax.fori_loop(0, 5, sigk, jnp.float32(0.0))
    plsc.store_scatter(sg1, [it // 8, it % 8],
                       jnp.full((16,), 1.0, jnp.float32) * tot)
    pltpu.sync_copy(sg1.at[pl.ds(0, 1), :], sig_sp.at[pl.ds(s, 1), :])
    plsc.subcore_barrier()
    pltpu.sync_copy(sig_sp, sg16)
    sigma = jnp.sum(plsc.load_gather(sg16, [it, jnp.zeros((16,), jnp.int32)])
                    ) * (1.0 / NTRAIN)

    def slot(k, cy):
        g = k * 16 + s

        @pl.when(g < NIB2)
        def _():
            off = g * IB2
            pltpu.sync_copy(anid.at[pl.ds(off, IB2)], ibuf)
            pltpu.sync_copy(ysoft.at[pl.ds(off, IB2), :], ysbuf)
            pltpu.sync_copy(
                ytrue.at[pl.ds(jnp.minimum(off, NTRAIN - IB2), IB2)], tyt)
            for q in range(NPL):
                pltpu.sync_copy(sm.at[q, pl.ds(off, IB2), :], sj[q])
            pltpu.sync_copy(norm.at[c].at[ibuf], nrows)

            def rs(v, cy2):
                i = v * 16 + it
                sab = jnp.zeros((16,), jnp.float32)
                for q in range(NPL):
                    for cc in range(CPL):
                        cv = jnp.full((16,), cc, jnp.int32)
                        sab = sab + jnp.abs(plsc.load_gather(sj[q], [i, cv]))
                sc_ = sigma / sab
                sc_ = jnp.where((sc_ > 1000.0) | (sc_ != sc_), 1.0, sc_)
                plsc.store_scatter(scl, [i], sc_)
                return cy2
            lax.fori_loop(0, IB2 // 16, rs, 0)

            trn = jnp.where(g < NTB2, jnp.float32(1.0), jnp.float32(0.0))
            for sp in range(2):
                p = 2 * c + sp
                pltpu.sync_copy(sm.at[p].at[ibuf], r1)

                def f(v):
                    k2 = v * 16 + it
                    i = k2 // PL
                    cc = k2 % PL
                    cls = jnp.minimum(cc + p * CPL, C - 1)
                    smv = plsc.load_gather(r1, [i, cc])
                    scv = plsc.load_gather(scl, [i])
                    ys = plsc.load_gather(ysbuf, [i, cls])
                    res = ys + scv * smv
                    yt = plsc.load_gather(tyt, [i])
                    oh = jnp.where(yt == cls, 1.0, 0.0)
                    ya = trn * oh + (1.0 - trn) * res
                    ya = jnp.where(cc < CPL, ya, 0.0)
                    nv = plsc.load_gather(nrows, [i])
                    plsc.store_scatter(r1, [i, cc], nv * ya)
                    plsc.store_scatter(r2, [i, cc], (1.0 - SM_ALPHA) * ya)
                _floop(IB2 * PL // 16, f)
                pltpu.sync_copy(r1, yfs.at[p].at[ibuf])
                pltpu.sync_copy(r2, lasts.at[p].at[ibuf])
        return cy
    lax.fori_loop(0, 8, slot, 0)

    @pl.when(s == 15)
    def _():
        for sp in range(2):
            p = 2 * c + sp
            pltpu.sync_copy(zpad, yfs.at[p, pl.ds(N, NP - N), :])
            pltpu.sync_copy(zpad, lasts.at[p, pl.ds(N, NP - N), :])


def _smooth_body(srce, dste, anid, norm, yfs, lasts,
                 yft, yfu, out,
                 acc, ebuf, sidx, didx, ibuf, nchunk,
                 achunk, cb1, cb2, zbuf):
    c = lax.axis_index("c")
    s = lax.axis_index("s")
    r0 = s * TN

    _fill2d(zbuf, UC, PL, 0.0)
    _fill2d(cb2, UC, PL, 0.0)

    def zacc(u, cy):
        pltpu.sync_copy(zbuf, acc.at[pl.ds(r0 + u * UC, UC), :])
        return cy
    lax.fori_loop(0, NUC, zacc, 0)
    plsc.subcore_barrier()

    ysrcs = [yfs] + [yft if l % 2 == 1 else yfu for l in range(1, NLAYERS)]
    ydsts = [yft if l % 2 == 0 else yfu for l in range(NLAYERS)]
    _prop_layers(c, s, SM_ALPHA, 0.0, 1.0, ysrcs, ydsts, lasts, yfu,
                 srce, dste, norm, acc, ebuf, sidx, didx, nchunk,
                 achunk, cb1, cb2, zbuf)

    def slot(k, cy):
        g = k * 16 + s

        @pl.when(g < NIB2)
        def _():
            off = g * IB2
            pltpu.sync_copy(anid.at[pl.ds(off, IB2)], ibuf)
            for sp in range(2):
                p = 2 * c + sp
                pltpu.sync_copy(yfu.at[p].at[ibuf], ebuf)
                pltpu.sync_copy(ebuf, out.at[p, pl.ds(off, IB2), :])
        return cy
    lax.fori_loop(0, 8, slot, 0)


def kernel(y_soft, y_true, edge_index, train_nid, val_nid, test_nid, n_nodes):
    del n_nodes
    src = edge_index[0]
    dst = edge_index[1]
    all_nid = jnp.concatenate([train_nid, val_nid, test_nid], axis=0)
    f32 = jnp.float32
    i32 = jnp.int32
    mesh = plsc.VectorSubcoreMesh(core_axis_name="c", subcore_axis_name="s")
    cparams = pltpu.CompilerParams(
        use_tc_tiling_on_sc=False, needs_layout_passes=False)

    prop_scratch = [
        pltpu.VMEM_SHARED((NP, PL), f32),
        pltpu.VMEM((EB, PL), f32),
        pltpu.VMEM((EB,), i32),
        pltpu.VMEM((EB,), i32),
    ]
    upd_scratch = [
        pltpu.VMEM((UC,), f32),
        pltpu.VMEM((UC, PL), f32),
        pltpu.VMEM((UC, PL), f32),
        pltpu.VMEM((UC, PL), f32),
        pltpu.VMEM((UC, PL), f32),
    ]

    k1 = pl.kernel(
        _corr_body,
        out_type=[
            jax.ShapeDtypeStruct((2, NP), f32),
            jax.ShapeDtypeStruct((NPL, NP, PL), f32),
            jax.ShapeDtypeStruct((NPL, NP, PL), f32),
            jax.ShapeDtypeStruct((NPL, NP, PL), f32),
            jax.ShapeDtypeStruct((NPL, NP, PL), f32),
        ],
        mesh=mesh,
        compiler_params=cparams,
        scratch_types=prop_scratch + [
            pltpu.VMEM((IB1,), i32),
            pltpu.VMEM((IB1,), i32),
            pltpu.VMEM((IB1, C), f32),
            pltpu.VMEM((IB1,), f32),
        ] + upd_scratch,
    )
    norm, sm, _, _, _ = k1(y_soft, y_true, train_nid, src, dst)

    k2 = pl.kernel(
        _assemble_body,
        out_type=[
            jax.ShapeDtypeStruct((NPL, NP, PL), f32),
            jax.ShapeDtypeStruct((NPL, NP, PL), f32),
        ],
        mesh=mesh,
        compiler_params=cparams,
        scratch_types=[
            pltpu.VMEM_SHARED((16, 8), f32),
            pltpu.VMEM((IB2, C), f32),
            pltpu.VMEM((IB2,), i32),
            pltpu.VMEM((IB2,), i32),
            pltpu.VMEM((IB2, PL), f32),
            pltpu.VMEM((IB2, PL), f32),
            pltpu.VMEM((IB2, PL), f32),
            pltpu.VMEM((IB2, PL), f32),
            pltpu.VMEM((IB2,), f32),
            pltpu.VMEM((IB2,), f32),
            pltpu.VMEM((IB2, PL), f32),
            pltpu.VMEM((IB2, PL), f32),
            pltpu.VMEM((16, 8), f32),
            pltpu.VMEM((16, 8), f32),
            pltpu.VMEM((NP - N, PL), f32),
        ],
    )
    yfs, lasts = k2(y_soft, y_true, all_nid, sm, norm)

    k3 = pl.kernel(
        _smooth_body,
        out_type=[
            jax.ShapeDtypeStruct((NPL, NP, PL), f32),
            jax.ShapeDtypeStruct((NPL, NP, PL), f32),
            jax.ShapeDtypeStruct((NPL, N, PL), f32),
        ],
        mesh=mesh,
        compiler_params=cparams,
        scratch_types=prop_scratch + [
            pltpu.VMEM((IB2,), i32),
        ] + upd_scratch,
    )
    _, _, out4 = k3(src, dst, all_nid, norm, yfs, lasts)

    return jnp.concatenate([out4[0, :, :CPL], out4[1, :, :CPL],
                            out4[2, :, :CPL], out4[3, :, :CPL]], axis=1)


# speedup vs baseline: 6.9690x; 1.1107x over previous
import jax
import jax.numpy as jnp
from jax import lax
from jax.experimental import pallas as pl
from jax.experimental.pallas import tpu as pltpu
from jax.experimental.pallas import tpu_sc as plsc

N, E, C = 100000, 1600000, 40
PL = 16
CPL = 10
NPL = 4
NP = 100096
TN = NP // 16
UC = 136
NUC = TN // UC
EB = 400
ET = E // 16
NEB = ET // EB
IB3 = 400
NIB3 = N // IB3
IB1 = 120
NTB1 = 60000 // IB1
SLOT1 = 32
IB2 = 800
NIB2 = N // IB2
NTB2 = 60000 // IB2
CORR_ALPHA, SM_ALPHA = 0.979, 0.756
NLAYERS = 10
NTRAIN = 60000


def _iota():
    return lax.iota(jnp.int32, 16)


def _rsqrt(x):
    i = plsc.bitcast(x, jnp.int32)
    y = plsc.bitcast(jnp.full((16,), 0x5F3759DF, jnp.int32) - (i >> 1),
                     jnp.float32)
    for _ in range(3):
        y = y * (1.5 - 0.5 * x * y * y)
    return y


def _floop(nvec, f):
    def body(v, carry):
        f(v)
        return carry
    lax.fori_loop(0, nvec, body, 0)


def _fill2d(ref, rows, width, val):
    v16 = jnp.full((16,), val, jnp.float32)

    def f(v):
        k = v * 16 + _iota()
        plsc.store_scatter(ref, [k // width, k % width], v16)
    _floop(rows * width // 16, f)


def _prop_layers(c, s, alpha, lo, hi, ysrcs, ydsts, lastb, final_dst,
                 srce, dste, norm, acc, ebuf0, ebuf1, sidx0, didx0,
                 sidx1, didx1, gsem0, gsem1, nchunk, achunk, cb1, zbuf):
    it = _iota()
    r0 = s * TN
    e0 = s * ET
    for l in range(NLAYERS):
        ysrc = ysrcs[l]
        ydst = ydsts[l]
        final = l == NLAYERS - 1
        for sp in range(2):
            p = 2 * c + sp

            pltpu.sync_copy(srce.at[pl.ds(e0, EB)], sidx0)
            pltpu.sync_copy(dste.at[pl.ds(e0, EB)], didx0)
            pltpu.async_copy(ysrc.at[p].at[sidx0], ebuf0, gsem0)

            def pair(g, cy):
                off1 = e0 + (2 * g + 1) * EB
                pltpu.sync_copy(srce.at[pl.ds(off1, EB)], sidx1)
                pltpu.sync_copy(dste.at[pl.ds(off1, EB)], didx1)
                pltpu.async_copy(ysrc.at[p].at[sidx1], ebuf1, gsem1)
                pltpu.make_async_copy(ysrc.at[p].at[sidx0], ebuf0,
                                      gsem0).wait()
                pltpu.sync_copy(ebuf0, acc.at[didx0], add=True)

                @pl.when(g < NEB // 2 - 1)
                def _():
                    off2 = e0 + (2 * g + 2) * EB
                    pltpu.sync_copy(srce.at[pl.ds(off2, EB)], sidx0)
                    pltpu.sync_copy(dste.at[pl.ds(off2, EB)], didx0)
                    pltpu.async_copy(ysrc.at[p].at[sidx0], ebuf0, gsem0)
                pltpu.make_async_copy(ysrc.at[p].at[sidx1], ebuf1,
                                      gsem1).wait()
                pltpu.sync_copy(ebuf1, acc.at[didx1], add=True)
                return cy
            lax.fori_loop(0, NEB // 2, pair, 0)
            plsc.subcore_barrier()

            def up(u, cy):
                rb = r0 + u * UC
                pltpu.sync_copy(acc.at[pl.ds(rb, UC), :], achunk)
                pltpu.sync_copy(zbuf, acc.at[pl.ds(rb, UC), :])
                pltpu.sync_copy(lastb.at[p, pl.ds(rb, UC), :], cb1)
                pltpu.sync_copy(norm.at[c, pl.ds(rb, UC)], nchunk)

                def f(v):
                    k2 = v * 16 + it
                    i = k2 // CPL
                    cc = k2 % CPL
                    a = plsc.load_gather(achunk, [i, cc])
                    lv = plsc.load_gather(cb1, [i, cc])
                    nv = plsc.load_gather(nchunk, [i])
                    y = jnp.clip(lv + alpha * nv * a, lo, hi)
                    outv = y if final else nv * y
                    plsc.store_scatter(achunk, [i, cc], outv)
                _floop(UC * CPL // 16, f)
                tgt = final_dst if final else ydst
                pltpu.sync_copy(achunk, tgt.at[p, pl.ds(rb, UC), :])
                return cy
            lax.fori_loop(0, NUC, up, 0)
            plsc.subcore_barrier()


def _corr_body(ysoft, ytrue, tnid, srce, dste,
               norm_o, sm_o, yfa, yfb, lastb,
               acc, ebuf0, ebuf1, sidx0, didx0, sidx1, didx1, gsem0, gsem1,
               ibuf, tyt, ysbuf, nrows, nchunk,
               achunk, cb1, zbuf):
    c = lax.axis_index("c")
    s = lax.axis_index("s")
    r0 = s * TN
    it = _iota()

    _fill2d(zbuf, UC, PL, 0.0)
    _fill2d(ebuf0, EB, PL, 1.0)

    def zacc(u, cy):
        pltpu.sync_copy(zbuf, acc.at[pl.ds(r0 + u * UC, UC), :])
        return cy
    lax.fori_loop(0, NUC, zacc, 0)
    plsc.subcore_barrier()

    def degb(b, cy):
        off = s * ET + b * EB
        pltpu.sync_copy(dste.at[pl.ds(off, EB)], didx0)
        pltpu.sync_copy(ebuf0, acc.at[didx0], add=True)
        return cy
    lax.fori_loop(0, NEB, degb, 0)
    plsc.subcore_barrier()

    def nrm(u, cy):
        rb = r0 + u * UC
        pltpu.sync_copy(acc.at[pl.ds(rb, UC), :], achunk)
        pltpu.sync_copy(zbuf, acc.at[pl.ds(rb, UC), :])

        def f(v):
            i = v * 16 + it
            m = i < UC
            d = plsc.load_gather(achunk,
                                 [jnp.minimum(i, UC - 1),
                                  jnp.zeros((16,), jnp.int32)])
            nv = _rsqrt(jnp.maximum(d, 1.0))
            plsc.store_scatter(nchunk, [i], nv, mask=m)
        _floop((UC + 15) // 16, f)
        pltpu.sync_copy(nchunk, norm_o.at[c, pl.ds(rb, UC)])
        return cy
    lax.fori_loop(0, NUC, nrm, 0)

    def zyf(u, cy):
        rb = r0 + u * UC
        for sp in range(2):
            p = 2 * c + sp
            pltpu.sync_copy(zbuf, yfa.at[p, pl.ds(rb, UC), :])
            pltpu.sync_copy(zbuf, lastb.at[p, pl.ds(rb, UC), :])
        return cy
    lax.fori_loop(0, NUC, zyf, 0)
    plsc.subcore_barrier()

    r1 = ebuf0.at[pl.ds(0, IB1), :]
    r2 = ebuf1.at[pl.ds(0, IB1), :]

    def slot(k, cy):
        g = k * 16 + s

        @pl.when(g < NTB1)
        def _():
            off = g * IB1
            pltpu.sync_copy(tnid.at[pl.ds(off, IB1)], ibuf)
            pltpu.sync_copy(ytrue.at[pl.ds(off, IB1)], tyt)
            pltpu.sync_copy(ysoft.at[pl.ds(off, IB1), :], ysbuf)
            pltpu.sync_copy(norm_o.at[c].at[ibuf], nrows)
            for sp in range(2):
                p = 2 * c + sp

                def f(v):
                    k2 = v * 16 + it
                    i = k2 // PL
                    cc = k2 % PL
                    cls = jnp.minimum(cc + p * CPL, C - 1)
                    yt = plsc.load_gather(tyt, [i])
                    oh = jnp.where(yt == cls, 1.0, 0.0)
                    ys = plsc.load_gather(ysbuf, [i, cls])
                    nv = plsc.load_gather(nrows, [i])
                    err = jnp.where(cc < CPL, oh - ys, 0.0)
                    plsc.store_scatter(r1, [i, cc], nv * err)
                    plsc.store_scatter(r2, [i, cc], (1.0 - CORR_ALPHA) * err)
                _floop(IB1 * PL // 16, f)
                pltpu.sync_copy(r1, yfa.at[p].at[ibuf])
                pltpu.sync_copy(r2, lastb.at[p].at[ibuf])
        return cy
    lax.fori_loop(0, SLOT1, slot, 0)
    plsc.subcore_barrier()

    ysrcs = [yfa if l % 2 == 0 else yfb for l in range(NLAYERS)]
    ydsts = [yfb if l % 2 == 0 else yfa for l in range(NLAYERS)]
    _prop_layers(c, s, CORR_ALPHA, -1.0, 1.0, ysrcs, ydsts, lastb, sm_o,
                 srce, dste, norm_o, acc, ebuf0, ebuf1, sidx0, didx0,
                 sidx1, didx1, gsem0, gsem1, nchunk, achunk, cb1, zbuf)


def _assemble_body(ysoft, ytrue, anid, sm, norm,
                   yfs, lasts,
                   sig_sp, ysbuf, tyt, ibuf, sj0, sj1, sj2, sj3, nrows, scl,
                   r1, r2, sg16, sg1, zpad):
    c = lax.axis_index("c")
    s = lax.axis_index("s")
    it = _iota()
    sj = [sj0, sj1, sj2, sj3]

    _fill2d(zpad, NP - N, PL, 0.0)

    def sigk(k, tot):
        g = k * 16 + s
        off = jnp.minimum(g, NTB2 - 1) * IB2
        pltpu.sync_copy(ytrue.at[pl.ds(off, IB2)], tyt)
        pltpu.sync_copy(ysoft.at[pl.ds(off, IB2), :], ysbuf)

        def f(v, a):
            k2 = v * 16 + it
            i = k2 // C
            cc = k2 % C
            ys = plsc.load_gather(ysbuf, [i, cc])
            yt = plsc.load_gather(tyt, [i])
            oh = jnp.where(yt == cc, 1.0, 0.0)
            return a + jnp.abs(oh - ys)
        a16 = lax.fori_loop(0, IB2 * C // 16, f,
                            jnp.zeros((16,), jnp.float32))
        return tot + jnp.where(g < NTB2, jnp.sum(a16), 0.0)
    tot = lax.fori_loop(0, 5, sigk, jnp.float32(0.0))
    plsc.store_scatter(sg1, [it // 8, it % 8],
                       jnp.full((16,), 1.0, jnp.float32) * tot)
    pltpu.sync_copy(sg1.at[pl.ds(0, 1), :], sig_sp.at[pl.ds(s, 1), :])
    plsc.subcore_barrier()
    pltpu.sync_copy(sig_sp, sg16)
    sigma = jnp.sum(plsc.load_gather(sg16, [it, jnp.zeros((16,), jnp.int32)])
                    ) * (1.0 / NTRAIN)

    def slot(k, cy):
        g = k * 16 + s

        @pl.when(g < NIB2)
        def _():
            off = g * IB2
            pltpu.sync_copy(anid.at[pl.ds(off, IB2)], ibuf)
            pltpu.sync_copy(ysoft.at[pl.ds(off, IB2), :], ysbuf)
            pltpu.sync_copy(
                ytrue.at[pl.ds(jnp.minimum(off, NTRAIN - IB2), IB2)], tyt)
            for q in range(NPL):
                pltpu.sync_copy(sm.at[q, pl.ds(off, IB2), :], sj[q])
            pltpu.sync_copy(norm.at[c].at[ibuf], nrows)

            def rs(v, cy2):
                i = v * 16 + it
                sab = jnp.zeros((16,), jnp.float32)
                for q in range(NPL):
                    for cc in range(CPL):
                        cv = jnp.full((16,), cc, jnp.int32)
                        sab = sab + jnp.abs(plsc.load_gather(sj[q], [i, cv]))
                sc_ = sigma / sab
                sc_ = jnp.where((sc_ > 1000.0) | (sc_ != sc_), 1.0, sc_)
                plsc.store_scatter(scl, [i], sc_)
                return cy2
            lax.fori_loop(0, IB2 // 16, rs, 0)

            trn = jnp.where(g < NTB2, jnp.float32(1.0), jnp.float32(0.0))
            for sp in range(2):
                p = 2 * c + sp
                pltpu.sync_copy(sm.at[p].at[ibuf], r1)

                def f(v):
                    k2 = v * 16 + it
                    i = k2 // PL
                    cc = k2 % PL
                    cls = jnp.minimum(cc + p * CPL, C - 1)
                    smv = plsc.load_gather(r1, [i, cc])
                    scv = plsc.load_gather(scl, [i])
                    ys = plsc.load_gather(ysbuf, [i, cls])
                    res = ys + scv * smv
                    yt = plsc.load_gather(tyt, [i])
                    oh = jnp.where(yt == cls, 1.0, 0.0)
                    ya = trn * oh + (1.0 - trn) * res
                    ya = jnp.where(cc < CPL, ya, 0.0)
                    nv = plsc.load_gather(nrows, [i])
                    plsc.store_scatter(r1, [i, cc], nv * ya)
                    plsc.store_scatter(r2, [i, cc], (1.0 - SM_ALPHA) * ya)
                _floop(IB2 * PL // 16, f)
                pltpu.sync_copy(r1, yfs.at[p].at[ibuf])
                pltpu.sync_copy(r2, lasts.at[p].at[ibuf])
        return cy
    lax.fori_loop(0, 8, slot, 0)

    @pl.when(s == 15)
    def _():
        for sp in range(2):
            p = 2 * c + sp
            pltpu.sync_copy(zpad, yfs.at[p, pl.ds(N, NP - N), :])
            pltpu.sync_copy(zpad, lasts.at[p, pl.ds(N, NP - N), :])


def _smooth_body(srce, dste, anid, norm, yfs, lasts,
                 yft, yfu, out,
                 acc, ebuf0, ebuf1, sidx0, didx0, sidx1, didx1, gsem0, gsem1,
                 ibuf, nchunk, achunk, cb1, zbuf):
    c = lax.axis_index("c")
    s = lax.axis_index("s")
    r0 = s * TN

    _fill2d(zbuf, UC, PL, 0.0)

    def zacc(u, cy):
        pltpu.sync_copy(zbuf, acc.at[pl.ds(r0 + u * UC, UC), :])
        return cy
    lax.fori_loop(0, NUC, zacc, 0)
    plsc.subcore_barrier()

    ysrcs = [yfs] + [yft if l % 2 == 1 else yfu for l in range(1, NLAYERS)]
    ydsts = [yft if l % 2 == 0 else yfu for l in range(NLAYERS)]
    _prop_layers(c, s, SM_ALPHA, 0.0, 1.0, ysrcs, ydsts, lasts, yfu,
                 srce, dste, norm, acc, ebuf0, ebuf1, sidx0, didx0,
                 sidx1, didx1, gsem0, gsem1, nchunk, achunk, cb1, zbuf)

    def slot(k, cy):
        g = k * 16 + s

        @pl.when(g < NIB3)
        def _():
            off = g * IB3
            pltpu.sync_copy(anid.at[pl.ds(off, IB3)], ibuf)
            for sp in range(2):
                p = 2 * c + sp
                pltpu.sync_copy(yfu.at[p].at[ibuf], ebuf0)
                pltpu.sync_copy(ebuf0, out.at[p, pl.ds(off, IB3), :])
        return cy
    lax.fori_loop(0, 16, slot, 0)


def kernel(y_soft, y_true, edge_index, train_nid, val_nid, test_nid, n_nodes):
    del n_nodes
    src = edge_index[0]
    dst = edge_index[1]
    all_nid = jnp.concatenate([train_nid, val_nid, test_nid], axis=0)
    f32 = jnp.float32
    i32 = jnp.int32
    mesh = plsc.VectorSubcoreMesh(core_axis_name="c", subcore_axis_name="s")
    cparams = pltpu.CompilerParams(
        use_tc_tiling_on_sc=False, needs_layout_passes=False)

    prop_scratch = [
        pltpu.VMEM_SHARED((NP, PL), f32),
        pltpu.VMEM((EB, PL), f32),
        pltpu.VMEM((EB, PL), f32),
        pltpu.VMEM((EB,), i32),
        pltpu.VMEM((EB,), i32),
        pltpu.VMEM((EB,), i32),
        pltpu.VMEM((EB,), i32),
        pltpu.SemaphoreType.DMA,
        pltpu.SemaphoreType.DMA,
    ]
    upd_scratch = [
        pltpu.VMEM((UC,), f32),
        pltpu.VMEM((UC, PL), f32),
        pltpu.VMEM((UC, PL), f32),
        pltpu.VMEM((UC, PL), f32),
    ]

    k1 = pl.kernel(
        _corr_body,
        out_type=[
            jax.ShapeDtypeStruct((2, NP), f32),
            jax.ShapeDtypeStruct((NPL, NP, PL), f32),
            jax.ShapeDtypeStruct((NPL, NP, PL), f32),
            jax.ShapeDtypeStruct((NPL, NP, PL), f32),
            jax.ShapeDtypeStruct((NPL, NP, PL), f32),
        ],
        mesh=mesh,
        compiler_params=cparams,
        scratch_types=prop_scratch + [
            pltpu.VMEM((IB1,), i32),
            pltpu.VMEM((IB1,), i32),
            pltpu.VMEM((IB1, C), f32),
            pltpu.VMEM((IB1,), f32),
        ] + upd_scratch,
    )
    norm, sm, _, _, _ = k1(y_soft, y_true, train_nid, src, dst)

    k2 = pl.kernel(
        _assemble_body,
        out_type=[
            jax.ShapeDtypeStruct((NPL, NP, PL), f32),
            jax.ShapeDtypeStruct((NPL, NP, PL), f32),
        ],
        mesh=mesh,
        compiler_params=cparams,
        scratch_types=[
            pltpu.VMEM_SHARED((16, 8), f32),
            pltpu.VMEM((IB2, C), f32),
            pltpu.VMEM((IB2,), i32),
            pltpu.VMEM((IB2,), i32),
            pltpu.VMEM((IB2, PL), f32),
            pltpu.VMEM((IB2, PL), f32),
            pltpu.VMEM((IB2, PL), f32),
            pltpu.VMEM((IB2, PL), f32),
            pltpu.VMEM((IB2,), f32),
            pltpu.VMEM((IB2,), f32),
            pltpu.VMEM((IB2, PL), f32),
            pltpu.VMEM((IB2, PL), f32),
            pltpu.VMEM((16, 8), f32),
            pltpu.VMEM((16, 8), f32),
            pltpu.VMEM((NP - N, PL), f32),
        ],
    )
    yfs, lasts = k2(y_soft, y_true, all_nid, sm, norm)

    k3 = pl.kernel(
        _smooth_body,
        out_type=[
            jax.ShapeDtypeStruct((NPL, NP, PL), f32),
            jax.ShapeDtypeStruct((NPL, NP, PL), f32),
            jax.ShapeDtypeStruct((NPL, N, PL), f32),
        ],
        mesh=mesh,
        compiler_params=cparams,
        scratch_types=prop_scratch + [
            pltpu.VMEM((IB3,), i32),
        ] + upd_scratch,
    )
    _, _, out4 = k3(src, dst, all_nid, norm, yfs, lasts)

    return jnp.concatenate([out4[0, :, :CPL], out4[1, :, :CPL],
                            out4[2, :, :CPL], out4[3, :, :CPL]], axis=1)


# speedup vs baseline: 7.2732x; 1.0437x over previous
import jax
import jax.numpy as jnp
from jax import lax
from jax.experimental import pallas as pl
from jax.experimental.pallas import tpu as pltpu
from jax.experimental.pallas import tpu_sc as plsc

N, E, C = 100000, 1600000, 40
PL = 16
CPL = 10
NPL = 4
NP = 100096
TN = NP // 16
UC1 = 184
NUC1 = TN // UC1
UC3 = 272
NUC3 = TN // UC3
EB = 400
ET = E // 16
NEB = ET // EB
IB3 = 400
NIB3 = N // IB3
IB1 = 120
NTB1 = 60000 // IB1
SLOT1 = 32
IB2 = 800
NIB2 = N // IB2
NTB2 = 60000 // IB2
CORR_ALPHA, SM_ALPHA = 0.979, 0.756
NLAYERS = 10
NTRAIN = 60000


def _iota():
    return lax.iota(jnp.int32, 16)


def _rsqrt(x):
    i = plsc.bitcast(x, jnp.int32)
    y = plsc.bitcast(jnp.full((16,), 0x5F3759DF, jnp.int32) - (i >> 1),
                     jnp.float32)
    for _ in range(3):
        y = y * (1.5 - 0.5 * x * y * y)
    return y


def _floop(nvec, f):
    def body(v, carry):
        f(v)
        return carry
    lax.fori_loop(0, nvec, body, 0)


def _fill2d(ref, rows, width, val):
    v16 = jnp.full((16,), val, jnp.float32)

    def f(v):
        k = v * 16 + _iota()
        plsc.store_scatter(ref, [k // width, k % width], v16)
    _floop(rows * width // 16, f)


def _prop_layers(c, s, alpha, lo, hi, ysrcs, ydsts, lastb, final_dst,
                 srce, dste, norm, acc, ebuf0, ebuf1, sidx0, didx0,
                 sidx1, didx1, gsem0, gsem1, nchunk, achunk, cb1, zbuf,
                 uc, nuc):
    it = _iota()
    r0 = s * TN
    e0 = s * ET
    for l in range(NLAYERS):
        ysrc = ysrcs[l]
        ydst = ydsts[l]
        final = l == NLAYERS - 1
        for sp in range(2):
            p = 2 * c + sp

            pltpu.sync_copy(srce.at[pl.ds(e0, EB)], sidx0)
            pltpu.sync_copy(dste.at[pl.ds(e0, EB)], didx0)
            pltpu.async_copy(ysrc.at[p].at[sidx0], ebuf0, gsem0)

            def pair(g, cy):
                off1 = e0 + (2 * g + 1) * EB
                pltpu.sync_copy(srce.at[pl.ds(off1, EB)], sidx1)
                pltpu.sync_copy(dste.at[pl.ds(off1, EB)], didx1)
                pltpu.async_copy(ysrc.at[p].at[sidx1], ebuf1, gsem1)
                pltpu.make_async_copy(ysrc.at[p].at[sidx0], ebuf0,
                                      gsem0).wait()
                pltpu.sync_copy(ebuf0, acc.at[didx0], add=True)

                @pl.when(g < NEB // 2 - 1)
                def _():
                    off2 = e0 + (2 * g + 2) * EB
                    pltpu.sync_copy(srce.at[pl.ds(off2, EB)], sidx0)
                    pltpu.sync_copy(dste.at[pl.ds(off2, EB)], didx0)
                    pltpu.async_copy(ysrc.at[p].at[sidx0], ebuf0, gsem0)
                pltpu.make_async_copy(ysrc.at[p].at[sidx1], ebuf1,
                                      gsem1).wait()
                pltpu.sync_copy(ebuf1, acc.at[didx1], add=True)
                return cy
            lax.fori_loop(0, NEB // 2, pair, 0)
            plsc.subcore_barrier()

            def up(u, cy):
                rb = r0 + u * uc
                pltpu.sync_copy(acc.at[pl.ds(rb, uc), :], achunk)
                pltpu.sync_copy(zbuf, acc.at[pl.ds(rb, uc), :])
                pltpu.sync_copy(lastb.at[p, pl.ds(rb, uc), :], cb1)
                pltpu.sync_copy(norm.at[c, pl.ds(rb, uc)], nchunk)

                def f(v):
                    k2 = v * 16 + it
                    i = k2 // CPL
                    cc = k2 % CPL
                    a = plsc.load_gather(achunk, [i, cc])
                    lv = plsc.load_gather(cb1, [i, cc])
                    nv = plsc.load_gather(nchunk, [i])
                    y = jnp.clip(lv + alpha * nv * a, lo, hi)
                    outv = y if final else nv * y
                    plsc.store_scatter(achunk, [i, cc], outv)
                _floop(uc * CPL // 16, f)
                tgt = final_dst if final else ydst
                pltpu.sync_copy(achunk, tgt.at[p, pl.ds(rb, uc), :])
                return cy
            lax.fori_loop(0, nuc, up, 0)
            plsc.subcore_barrier()


def _corr_body(ysoft, ytrue, tnid, srce, dste,
               norm_o, sm_o, yfa, yfb, lastb,
               acc, ebuf0, ebuf1, sidx0, didx0, sidx1, didx1, gsem0, gsem1,
               ibuf, tyt, ysbuf, nrows, nchunk,
               achunk, cb1, zbuf):
    c = lax.axis_index("c")
    s = lax.axis_index("s")
    r0 = s * TN
    it = _iota()

    _fill2d(zbuf, UC1, PL, 0.0)
    _fill2d(ebuf0, EB, PL, 1.0)

    def zacc(u, cy):
        pltpu.sync_copy(zbuf, acc.at[pl.ds(r0 + u * UC1, UC1), :])
        return cy
    lax.fori_loop(0, NUC1, zacc, 0)
    plsc.subcore_barrier()

    def degb(b, cy):
        off = s * ET + b * EB
        pltpu.sync_copy(dste.at[pl.ds(off, EB)], didx0)
        pltpu.sync_copy(ebuf0, acc.at[didx0], add=True)
        return cy
    lax.fori_loop(0, NEB, degb, 0)
    plsc.subcore_barrier()

    def nrm(u, cy):
        rb = r0 + u * UC1
        pltpu.sync_copy(acc.at[pl.ds(rb, UC1), :], achunk)
        pltpu.sync_copy(zbuf, acc.at[pl.ds(rb, UC1), :])

        def f(v):
            i = v * 16 + it
            m = i < UC1
            d = plsc.load_gather(achunk,
                                 [jnp.minimum(i, UC1 - 1),
                                  jnp.zeros((16,), jnp.int32)])
            nv = _rsqrt(jnp.maximum(d, 1.0))
            plsc.store_scatter(nchunk, [i], nv, mask=m)
        _floop((UC1 + 15) // 16, f)
        pltpu.sync_copy(nchunk, norm_o.at[c, pl.ds(rb, UC1)])
        return cy
    lax.fori_loop(0, NUC1, nrm, 0)

    def zyf(u, cy):
        rb = r0 + u * UC1
        for sp in range(2):
            p = 2 * c + sp
            pltpu.sync_copy(zbuf, yfa.at[p, pl.ds(rb, UC1), :])
            pltpu.sync_copy(zbuf, lastb.at[p, pl.ds(rb, UC1), :])
        return cy
    lax.fori_loop(0, NUC1, zyf, 0)
    plsc.subcore_barrier()

    r1 = ebuf0.at[pl.ds(0, IB1), :]
    r2 = ebuf1.at[pl.ds(0, IB1), :]

    def slot(k, cy):
        g = k * 16 + s

        @pl.when(g < NTB1)
        def _():
            off = g * IB1
            pltpu.sync_copy(tnid.at[pl.ds(off, IB1)], ibuf)
            pltpu.sync_copy(ytrue.at[pl.ds(off, IB1)], tyt)
            pltpu.sync_copy(ysoft.at[pl.ds(off, IB1), :], ysbuf)
            pltpu.sync_copy(norm_o.at[c].at[ibuf], nrows)
            for sp in range(2):
                p = 2 * c + sp

                def f(v):
                    k2 = v * 16 + it
                    i = k2 // PL
                    cc = k2 % PL
                    cls = jnp.minimum(cc + p * CPL, C - 1)
                    yt = plsc.load_gather(tyt, [i])
                    oh = jnp.where(yt == cls, 1.0, 0.0)
                    ys = plsc.load_gather(ysbuf, [i, cls])
                    nv = plsc.load_gather(nrows, [i])
                    err = jnp.where(cc < CPL, oh - ys, 0.0)
                    plsc.store_scatter(r1, [i, cc], nv * err)
                    plsc.store_scatter(r2, [i, cc], (1.0 - CORR_ALPHA) * err)
                _floop(IB1 * PL // 16, f)
                pltpu.sync_copy(r1, yfa.at[p].at[ibuf])
                pltpu.sync_copy(r2, lastb.at[p].at[ibuf])
        return cy
    lax.fori_loop(0, SLOT1, slot, 0)
    plsc.subcore_barrier()

    ysrcs = [yfa if l % 2 == 0 else yfb for l in range(NLAYERS)]
    ydsts = [yfb if l % 2 == 0 else yfa for l in range(NLAYERS)]
    _prop_layers(c, s, CORR_ALPHA, -1.0, 1.0, ysrcs, ydsts, lastb, sm_o,
                 srce, dste, norm_o, acc, ebuf0, ebuf1, sidx0, didx0,
                 sidx1, didx1, gsem0, gsem1, nchunk, achunk, cb1, zbuf,
                 UC1, NUC1)


def _assemble_body(ysoft, ytrue, anid, sm, norm,
                   yfs, lasts,
                   sig_sp, ysbuf, tyt, ibuf, sj0, sj1, sj2, sj3, nrows, scl,
                   r1, r2, sg16, sg1, zpad):
    c = lax.axis_index("c")
    s = lax.axis_index("s")
    it = _iota()
    sj = [sj0, sj1, sj2, sj3]

    _fill2d(zpad, NP - N, PL, 0.0)

    def sigk(k, tot):
        g = k * 16 + s
        off = jnp.minimum(g, NTB2 - 1) * IB2
        pltpu.sync_copy(ytrue.at[pl.ds(off, IB2)], tyt)
        pltpu.sync_copy(ysoft.at[pl.ds(off, IB2), :], ysbuf)

        def f(v, a):
            k2 = v * 16 + it
            i = k2 // C
            cc = k2 % C
            ys = plsc.load_gather(ysbuf, [i, cc])
            yt = plsc.load_gather(tyt, [i])
            oh = jnp.where(yt == cc, 1.0, 0.0)
            return a + jnp.abs(oh - ys)
        a16 = lax.fori_loop(0, IB2 * C // 16, f,
                            jnp.zeros((16,), jnp.float32))
        return tot + jnp.where(g < NTB2, jnp.sum(a16), 0.0)
    tot = lax.fori_loop(0, 5, sigk, jnp.float32(0.0))
    plsc.store_scatter(sg1, [it // 8, it % 8],
                       jnp.full((16,), 1.0, jnp.float32) * tot)
    pltpu.sync_copy(sg1.at[pl.ds(0, 1), :], sig_sp.at[pl.ds(s, 1), :])
    plsc.subcore_barrier()
    pltpu.sync_copy(sig_sp, sg16)
    sigma = jnp.sum(plsc.load_gather(sg16, [it, jnp.zeros((16,), jnp.int32)])
                    ) * (1.0 / NTRAIN)

    def slot(k, cy):
        g = k * 16 + s

        @pl.when(g < NIB2)
        def _():
            off = g * IB2
            pltpu.sync_copy(anid.at[pl.ds(off, IB2)], ibuf)
            pltpu.sync_copy(ysoft.at[pl.ds(off, IB2), :], ysbuf)
            pltpu.sync_copy(
                ytrue.at[pl.ds(jnp.minimum(off, NTRAIN - IB2), IB2)], tyt)
            for q in range(NPL):
                pltpu.sync_copy(sm.at[q, pl.ds(off, IB2), :], sj[q])
            pltpu.sync_copy(norm.at[c].at[ibuf], nrows)

            def rs(v, cy2):
                i = v * 16 + it
                sab = jnp.zeros((16,), jnp.float32)
                for q in range(NPL):
                    for cc in range(CPL):
                        cv = jnp.full((16,), cc, jnp.int32)
                        sab = sab + jnp.abs(plsc.load_gather(sj[q], [i, cv]))
                sc_ = sigma / sab
                sc_ = jnp.where((sc_ > 1000.0) | (sc_ != sc_), 1.0, sc_)
                plsc.store_scatter(scl, [i], sc_)
                return cy2
            lax.fori_loop(0, IB2 // 16, rs, 0)

            trn = jnp.where(g < NTB2, jnp.float32(1.0), jnp.float32(0.0))
            for sp in range(2):
                p = 2 * c + sp
                pltpu.sync_copy(sm.at[p].at[ibuf], r1)

                def f(v):
                    k2 = v * 16 + it
                    i = k2 // PL
                    cc = k2 % PL
                    cls = jnp.minimum(cc + p * CPL, C - 1)
                    smv = plsc.load_gather(r1, [i, cc])
                    scv = plsc.load_gather(scl, [i])
                    ys = plsc.load_gather(ysbuf, [i, cls])
                    res = ys + scv * smv
                    yt = plsc.load_gather(tyt, [i])
                    oh = jnp.where(yt == cls, 1.0, 0.0)
                    ya = trn * oh + (1.0 - trn) * res
                    ya = jnp.where(cc < CPL, ya, 0.0)
                    nv = plsc.load_gather(nrows, [i])
                    plsc.store_scatter(r1, [i, cc], nv * ya)
                    plsc.store_scatter(r2, [i, cc], (1.0 - SM_ALPHA) * ya)
                _floop(IB2 * PL // 16, f)
                pltpu.sync_copy(r1, yfs.at[p].at[ibuf])
                pltpu.sync_copy(r2, lasts.at[p].at[ibuf])
        return cy
    lax.fori_loop(0, 8, slot, 0)

    @pl.when(s == 15)
    def _():
        for sp in range(2):
            p = 2 * c + sp
            pltpu.sync_copy(zpad, yfs.at[p, pl.ds(N, NP - N), :])
            pltpu.sync_copy(zpad, lasts.at[p, pl.ds(N, NP - N), :])


def _smooth_body(srce, dste, anid, norm, yfs, lasts,
                 yft, yfu, out,
                 acc, ebuf0, ebuf1, sidx0, didx0, sidx1, didx1, gsem0, gsem1,
                 ibuf, nchunk, achunk, cb1, zbuf):
    c = lax.axis_index("c")
    s = lax.axis_index("s")
    r0 = s * TN

    _fill2d(zbuf, UC3, PL, 0.0)

    def zacc(u, cy):
        pltpu.sync_copy(zbuf, acc.at[pl.ds(r0 + u * UC3, UC3), :])
        return cy
    lax.fori_loop(0, NUC3, zacc, 0)
    plsc.subcore_barrier()

    ysrcs = [yfs] + [yft if l % 2 == 1 else yfu for l in range(1, NLAYERS)]
    ydsts = [yft if l % 2 == 0 else yfu for l in range(NLAYERS)]
    _prop_layers(c, s, SM_ALPHA, 0.0, 1.0, ysrcs, ydsts, lasts, yfu,
                 srce, dste, norm, acc, ebuf0, ebuf1, sidx0, didx0,
                 sidx1, didx1, gsem0, gsem1, nchunk, achunk, cb1, zbuf,
                 UC3, NUC3)

    def slot(k, cy):
        g = k * 16 + s

        @pl.when(g < NIB3)
        def _():
            off = g * IB3
            pltpu.sync_copy(anid.at[pl.ds(off, IB3)], ibuf)
            for sp in range(2):
                p = 2 * c + sp
                pltpu.sync_copy(yfu.at[p].at[ibuf], ebuf0)
                pltpu.sync_copy(ebuf0, out.at[p, pl.ds(off, IB3), :])
        return cy
    lax.fori_loop(0, 16, slot, 0)


def kernel(y_soft, y_true, edge_index, train_nid, val_nid, test_nid, n_nodes):
    del n_nodes
    src = edge_index[0]
    dst = edge_index[1]
    all_nid = jnp.concatenate([train_nid, val_nid, test_nid], axis=0)
    f32 = jnp.float32
    i32 = jnp.int32
    mesh = plsc.VectorSubcoreMesh(core_axis_name="c", subcore_axis_name="s")
    cparams = pltpu.CompilerParams(
        use_tc_tiling_on_sc=False, needs_layout_passes=False)

    prop_scratch = [
        pltpu.VMEM_SHARED((NP, PL), f32),
        pltpu.VMEM((EB, PL), f32),
        pltpu.VMEM((EB, PL), f32),
        pltpu.VMEM((EB,), i32),
        pltpu.VMEM((EB,), i32),
        pltpu.VMEM((EB,), i32),
        pltpu.VMEM((EB,), i32),
        pltpu.SemaphoreType.DMA,
        pltpu.SemaphoreType.DMA,
    ]
    def upd_scratch(uc):
        return [
            pltpu.VMEM((uc,), f32),
            pltpu.VMEM((uc, PL), f32),
            pltpu.VMEM((uc, PL), f32),
            pltpu.VMEM((uc, PL), f32),
        ]

    k1 = pl.kernel(
        _corr_body,
        out_type=[
            jax.ShapeDtypeStruct((2, NP), f32),
            jax.ShapeDtypeStruct((NPL, NP, PL), f32),
            jax.ShapeDtypeStruct((NPL, NP, PL), f32),
            jax.ShapeDtypeStruct((NPL, NP, PL), f32),
            jax.ShapeDtypeStruct((NPL, NP, PL), f32),
        ],
        mesh=mesh,
        compiler_params=cparams,
        scratch_types=prop_scratch + [
            pltpu.VMEM((IB1,), i32),
            pltpu.VMEM((IB1,), i32),
            pltpu.VMEM((IB1, C), f32),
            pltpu.VMEM((IB1,), f32),
        ] + upd_scratch(UC1),
    )
    norm, sm, _, _, _ = k1(y_soft, y_true, train_nid, src, dst)

    k2 = pl.kernel(
        _assemble_body,
        out_type=[
            jax.ShapeDtypeStruct((NPL, NP, PL), f32),
            jax.ShapeDtypeStruct((NPL, NP, PL), f32),
        ],
        mesh=mesh,
        compiler_params=cparams,
        scratch_types=[
            pltpu.VMEM_SHARED((16, 8), f32),
            pltpu.VMEM((IB2, C), f32),
            pltpu.VMEM((IB2,), i32),
            pltpu.VMEM((IB2,), i32),
            pltpu.VMEM((IB2, PL), f32),
            pltpu.VMEM((IB2, PL), f32),
            pltpu.VMEM((IB2, PL), f32),
            pltpu.VMEM((IB2, PL), f32),
            pltpu.VMEM((IB2,), f32),
            pltpu.VMEM((IB2,), f32),
            pltpu.VMEM((IB2, PL), f32),
            pltpu.VMEM((IB2, PL), f32),
            pltpu.VMEM((16, 8), f32),
            pltpu.VMEM((16, 8), f32),
            pltpu.VMEM((NP - N, PL), f32),
        ],
    )
    yfs, lasts = k2(y_soft, y_true, all_nid, sm, norm)

    k3 = pl.kernel(
        _smooth_body,
        out_type=[
            jax.ShapeDtypeStruct((NPL, NP, PL), f32),
            jax.ShapeDtypeStruct((NPL, NP, PL), f32),
            jax.ShapeDtypeStruct((NPL, N, PL), f32),
        ],
        mesh=mesh,
        compiler_params=cparams,
        scratch_types=prop_scratch + [
            pltpu.VMEM((IB3,), i32),
        ] + upd_scratch(UC3),
    )
    _, _, out4 = k3(src, dst, all_nid, norm, yfs, lasts)

    return jnp.concatenate([out4[0, :, :CPL], out4[1, :, :CPL],
                            out4[2, :, :CPL], out4[3, :, :CPL]], axis=1)
